# baseline (speedup 1.0000x reference)
import functools

import jax
import jax.numpy as jnp
from jax import lax
from jax.experimental import pallas as pl
from jax.experimental.pallas import tpu as pltpu
from jax.experimental.pallas import tpu_sc as plsc

_N = 10000
_E = 320000
_F = 128
_NC = 2
_NS = 16
_NW = _NC * _NS
_EPW = _E // _NW
_B = 80
_CH = _EPW // _B
_G = 25
_NG = _CH // _G
_RPT = _N // _NS

_mesh = plsc.VectorSubcoreMesh(core_axis_name="c", subcore_axis_name="s")


def _sc_edge_aggregate(y, src_r, dst_r, zrows):
    @functools.partial(
        pl.kernel,
        out_type=jax.ShapeDtypeStruct((_NC, _NS, _RPT, _F), jnp.float32),
        mesh=_mesh,
        scratch_types=[
            pltpu.VMEM_SHARED((_N, _F), jnp.float32),
            pltpu.VMEM((_G, _B), jnp.int32),
            pltpu.VMEM((_G, _B), jnp.int32),
            pltpu.VMEM((_B, _F), jnp.float32),
            pltpu.VMEM((_B, _F), jnp.float32),
            pltpu.SemaphoreType.DMA,
            pltpu.SemaphoreType.DMA,
        ],
    )
    def k(y_hbm, src_hbm, dst_hbm, z_hbm, out_hbm, acc, src_v, dst_v,
          buf0, buf1, sem0, sem1):
        c = lax.axis_index("c")
        s = lax.axis_index("s")
        wid = s * _NC + c
        pltpu.sync_copy(z_hbm, acc.at[pl.ds(s * _RPT, _RPT)])
        plsc.subcore_barrier()

        def group(g, carry):
            pltpu.sync_copy(src_hbm.at[wid * _NG + g], src_v)
            pltpu.sync_copy(dst_hbm.at[wid * _NG + g], dst_v)

            def gather(j, buf, sem):
                return pltpu.make_async_copy(y_hbm.at[src_v.at[j]], buf, sem)

            def scat(j, buf):
                pltpu.sync_copy(buf, acc.at[dst_v.at[j]], add=True)

            def body(j, c2):
                gather(j, buf0, sem0).start()
                gather(j, buf0, sem0).wait()
                scat(j, buf0)
                return c2

            lax.fori_loop(0, _G, body, 0)
            return carry

        lax.fori_loop(0, _NG, group, 0)
        plsc.subcore_barrier()
        pltpu.sync_copy(acc.at[pl.ds(s * _RPT, _RPT)], out_hbm.at[c, s])

    return k(y, src_r, dst_r, zrows).reshape(_NC, _N, _F)


def _dinv_block(deg):
    return lax.rsqrt(deg)


_GRID = 10
_BR = _N // _GRID


def _tc_first(x, W1, deg):
    def body(x_ref, w_ref, deg_ref, y_ref):
        dinv = _dinv_block(deg_ref[...])
        y_ref[...] = jnp.dot(x_ref[...], w_ref[...],
                             preferred_element_type=jnp.float32) * dinv

    return pl.pallas_call(
        body,
        grid=(_GRID,),
        in_specs=[
            pl.BlockSpec((_BR, _F), lambda i: (i, 0)),
            pl.BlockSpec((_F, _F), lambda i: (0, 0)),
            pl.BlockSpec((_BR, 1), lambda i: (i, 0)),
        ],
        out_specs=pl.BlockSpec((_BR, _F), lambda i: (i, 0)),
        out_shape=jax.ShapeDtypeStruct((_N, _F), jnp.float32),
    )(x, W1, deg)


def _tc_mid(agga, aggb, y1, deg, W2, b1):
    def body(aa_ref, ab_ref, y1_ref, deg_ref, w_ref, b_ref, y2_ref):
        dinv = _dinv_block(deg_ref[...])
        h1 = jax.nn.sigmoid(
            (aa_ref[...] + ab_ref[...] + y1_ref[...]) * dinv + b_ref[...])
        y2_ref[...] = jnp.dot(h1, w_ref[...],
                              preferred_element_type=jnp.float32) * dinv

    return pl.pallas_call(
        body,
        grid=(_GRID,),
        in_specs=[
            pl.BlockSpec((_BR, _F), lambda i: (i, 0)),
            pl.BlockSpec((_BR, _F), lambda i: (i, 0)),
            pl.BlockSpec((_BR, _F), lambda i: (i, 0)),
            pl.BlockSpec((_BR, 1), lambda i: (i, 0)),
            pl.BlockSpec((_F, _F), lambda i: (0, 0)),
            pl.BlockSpec((1, _F), lambda i: (0, 0)),
        ],
        out_specs=pl.BlockSpec((_BR, _F), lambda i: (i, 0)),
        out_shape=jax.ShapeDtypeStruct((_N, _F), jnp.float32),
    )(agga, aggb, y1, deg, W2, b1)


def _tc_last(agga, aggb, y2, deg, b2, batch_r,
             Wil, bil, Whl1, bhl1, Wol, bol):
    def body(aa_ref, ab_ref, y2_ref, deg_ref, b2_ref, batch_ref,
             wil_ref, bil_ref, whl_ref, bhl_ref, wol_ref, bol_ref, out_ref):
        dinv = _dinv_block(deg_ref[...])
        h2 = jax.nn.relu(
            (aa_ref[...] + ab_ref[...] + y2_ref[...]) * dinv + b2_ref[...])
        gid = lax.broadcasted_iota(jnp.int32, (64, _N), 0)
        m = (batch_ref[...] == gid).astype(jnp.float32)
        sums = jnp.dot(m, h2, preferred_element_type=jnp.float32)
        cnts = jnp.sum(m, axis=1, keepdims=True)
        pooled = sums / jnp.maximum(cnts, 1.0)
        o = jax.nn.sigmoid(jnp.dot(pooled, wil_ref[...],
                                   preferred_element_type=jnp.float32)
                           + bil_ref[...])
        o = jax.nn.relu(jnp.dot(o, whl_ref[...],
                                preferred_element_type=jnp.float32)
                        + bhl_ref[...])
        out_ref[...] = (jnp.dot(o, wol_ref[...],
                                preferred_element_type=jnp.float32)
                        + bol_ref[...])

    return pl.pallas_call(
        body,
        out_shape=jax.ShapeDtypeStruct((64, 1), jnp.float32),
    )(agga, aggb, y2, deg, b2, batch_r, Wil, bil, Whl1, bhl1, Wol, bol)


def kernel(x, edge_index, batch, W1, b1, W2, b2, Wil, bil, Whl1, bhl1, Wol, bol):
    src_r = edge_index[0].reshape(_NW * _NG, _G, _B)
    dst_r = edge_index[1].reshape(_NW * _NG, _G, _B)
    z128 = jnp.zeros((_RPT, _F), jnp.float32)

    ones = jnp.ones((_N, _F), jnp.float32)
    cnt = _sc_edge_aggregate(ones, src_r, dst_r, z128)
    deg = (cnt[0, :, :1] + cnt[1, :, :1] + 1.0)
    y1 = _tc_first(x, W1, deg)
    agg1 = _sc_edge_aggregate(y1, src_r, dst_r, z128)
    y2 = _tc_mid(agg1[0], agg1[1], y1, deg, W2, b1.reshape(1, _F))
    agg2 = _sc_edge_aggregate(y2, src_r, dst_r, z128)
    return _tc_last(agg2[0], agg2[1], y2, deg, b2.reshape(1, _F),
                    batch.reshape(1, _N).astype(jnp.int32),
                    Wil, bil.reshape(1, 64), Whl1, bhl1.reshape(1, 16),
                    Wol, bol.reshape(1, 1))


# speedup vs baseline: 1.2271x; 1.2271x over previous
import functools

import jax
import jax.numpy as jnp
from jax import lax
from jax.experimental import pallas as pl
from jax.experimental.pallas import tpu as pltpu
from jax.experimental.pallas import tpu_sc as plsc

_N = 10000
_E = 320000
_F = 128
_NC = 2
_NS = 16
_NW = _NC * _NS
_EPW = _E // _NW
_B = 80
_CH = _EPW // _B
_G = 25
_NG = _CH // _G
_RPT = _N // _NS

_mesh = plsc.VectorSubcoreMesh(core_axis_name="c", subcore_axis_name="s")


def _sc_edge_aggregate(y, src_r, dst_r, zrows):
    @functools.partial(
        pl.kernel,
        out_type=jax.ShapeDtypeStruct((_NC, _NS, _RPT, _F), jnp.float32),
        mesh=_mesh,
        scratch_types=[
            pltpu.VMEM_SHARED((_N, _F), jnp.float32),
            pltpu.VMEM((_G, _B), jnp.int32),
            pltpu.VMEM((_G, _B), jnp.int32),
            pltpu.VMEM((_B, _F), jnp.float32),
            pltpu.VMEM((_B, _F), jnp.float32),
            pltpu.SemaphoreType.DMA,
            pltpu.SemaphoreType.DMA,
        ],
    )
    def k(y_hbm, src_hbm, dst_hbm, z_hbm, out_hbm, acc, src_v, dst_v,
          buf0, buf1, sem0, sem1):
        c = lax.axis_index("c")
        s = lax.axis_index("s")
        wid = s * _NC + c
        pltpu.sync_copy(z_hbm, acc.at[pl.ds(s * _RPT, _RPT)])
        plsc.subcore_barrier()

        def group(g, carry):
            pltpu.sync_copy(src_hbm.at[wid * _NG + g], src_v)
            pltpu.sync_copy(dst_hbm.at[wid * _NG + g], dst_v)

            def gather(j, buf, sem):
                return pltpu.make_async_copy(y_hbm.at[src_v.at[j]], buf, sem)

            def scat(j, buf):
                pltpu.sync_copy(buf, acc.at[dst_v.at[j]], add=True)

            def body(j, c2):
                gather(j, buf0, sem0).start()
                gather(j, buf0, sem0).wait()
                scat(j, buf0)
                return c2

            lax.fori_loop(0, _G, body, 0)
            return carry

        lax.fori_loop(0, _NG, group, 0)
        plsc.subcore_barrier()
        pltpu.sync_copy(acc.at[pl.ds(s * _RPT, _RPT)], out_hbm.at[c, s])

    return k(y, src_r, dst_r, zrows).reshape(_NC, _N, _F)


def _sc_degree_cnt(ones80, dst_r, zrows):
    @functools.partial(
        pl.kernel,
        out_type=jax.ShapeDtypeStruct((_NC, _NS, _RPT, _F), jnp.float32),
        mesh=_mesh,
        scratch_types=[
            pltpu.VMEM_SHARED((_N, _F), jnp.float32),
            pltpu.VMEM((_G, _B), jnp.int32),
            pltpu.VMEM((_B, _F), jnp.float32),
        ],
    )
    def k(ones_hbm, dst_hbm, z_hbm, out_hbm, acc, dst_v, buf):
        c = lax.axis_index("c")
        s = lax.axis_index("s")
        wid = s * _NC + c
        pltpu.sync_copy(z_hbm, acc.at[pl.ds(s * _RPT, _RPT)])
        pltpu.sync_copy(ones_hbm, buf)
        plsc.subcore_barrier()

        def group(g, carry):
            pltpu.sync_copy(dst_hbm.at[wid * _NG + g], dst_v)

            def body(j, c2):
                pltpu.sync_copy(buf, acc.at[dst_v.at[j]], add=True)
                return c2

            lax.fori_loop(0, _G, body, 0)
            return carry

        lax.fori_loop(0, _NG, group, 0)
        plsc.subcore_barrier()
        pltpu.sync_copy(acc.at[pl.ds(s * _RPT, _RPT)], out_hbm.at[c, s])

    return k(ones80, dst_r, zrows).reshape(_NC, _N, _F)


def _dinv_block(deg):
    return lax.rsqrt(deg)


_GRID = 10
_BR = _N // _GRID


def _tc_first(x, W1, deg):
    def body(x_ref, w_ref, deg_ref, y_ref):
        dinv = _dinv_block(deg_ref[...])
        y_ref[...] = jnp.dot(x_ref[...], w_ref[...],
                             preferred_element_type=jnp.float32) * dinv

    return pl.pallas_call(
        body,
        grid=(_GRID,),
        in_specs=[
            pl.BlockSpec((_BR, _F), lambda i: (i, 0)),
            pl.BlockSpec((_F, _F), lambda i: (0, 0)),
            pl.BlockSpec((_BR, 1), lambda i: (i, 0)),
        ],
        out_specs=pl.BlockSpec((_BR, _F), lambda i: (i, 0)),
        out_shape=jax.ShapeDtypeStruct((_N, _F), jnp.float32),
    )(x, W1, deg)


def _tc_mid(agga, aggb, y1, deg, W2, b1):
    def body(aa_ref, ab_ref, y1_ref, deg_ref, w_ref, b_ref, y2_ref):
        dinv = _dinv_block(deg_ref[...])
        h1 = jax.nn.sigmoid(
            (aa_ref[...] + ab_ref[...] + y1_ref[...]) * dinv + b_ref[...])
        y2_ref[...] = jnp.dot(h1, w_ref[...],
                              preferred_element_type=jnp.float32) * dinv

    return pl.pallas_call(
        body,
        grid=(_GRID,),
        in_specs=[
            pl.BlockSpec((_BR, _F), lambda i: (i, 0)),
            pl.BlockSpec((_BR, _F), lambda i: (i, 0)),
            pl.BlockSpec((_BR, _F), lambda i: (i, 0)),
            pl.BlockSpec((_BR, 1), lambda i: (i, 0)),
            pl.BlockSpec((_F, _F), lambda i: (0, 0)),
            pl.BlockSpec((1, _F), lambda i: (0, 0)),
        ],
        out_specs=pl.BlockSpec((_BR, _F), lambda i: (i, 0)),
        out_shape=jax.ShapeDtypeStruct((_N, _F), jnp.float32),
    )(agga, aggb, y1, deg, W2, b1)


def _tc_last(agga, aggb, y2, deg, b2, batch_r,
             Wil, bil, Whl1, bhl1, Wol, bol):
    def body(aa_ref, ab_ref, y2_ref, deg_ref, b2_ref, batch_ref,
             wil_ref, bil_ref, whl_ref, bhl_ref, wol_ref, bol_ref, out_ref):
        dinv = _dinv_block(deg_ref[...])
        h2 = jax.nn.relu(
            (aa_ref[...] + ab_ref[...] + y2_ref[...]) * dinv + b2_ref[...])
        gid = lax.broadcasted_iota(jnp.int32, (64, _N), 0)
        m = (batch_ref[...] == gid).astype(jnp.float32)
        sums = jnp.dot(m, h2, preferred_element_type=jnp.float32)
        cnts = jnp.sum(m, axis=1, keepdims=True)
        pooled = sums / jnp.maximum(cnts, 1.0)
        o = jax.nn.sigmoid(jnp.dot(pooled, wil_ref[...],
                                   preferred_element_type=jnp.float32)
                           + bil_ref[...])
        o = jax.nn.relu(jnp.dot(o, whl_ref[...],
                                preferred_element_type=jnp.float32)
                        + bhl_ref[...])
        out_ref[...] = (jnp.dot(o, wol_ref[...],
                                preferred_element_type=jnp.float32)
                        + bol_ref[...])

    return pl.pallas_call(
        body,
        out_shape=jax.ShapeDtypeStruct((64, 1), jnp.float32),
    )(agga, aggb, y2, deg, b2, batch_r, Wil, bil, Whl1, bhl1, Wol, bol)


def kernel(x, edge_index, batch, W1, b1, W2, b2, Wil, bil, Whl1, bhl1, Wol, bol):
    src_r = edge_index[0].reshape(_NW * _NG, _G, _B)
    dst_r = edge_index[1].reshape(_NW * _NG, _G, _B)
    z128 = jnp.zeros((_RPT, _F), jnp.float32)

    ones80 = jnp.ones((_B, _F), jnp.float32)
    cnt = _sc_degree_cnt(ones80, dst_r, z128)
    deg = (cnt[0, :, :1] + cnt[1, :, :1] + 1.0)
    y1 = _tc_first(x, W1, deg)
    agg1 = _sc_edge_aggregate(y1, src_r, dst_r, z128)
    y2 = _tc_mid(agg1[0], agg1[1], y1, deg, W2, b1.reshape(1, _F))
    agg2 = _sc_edge_aggregate(y2, src_r, dst_r, z128)
    return _tc_last(agg2[0], agg2[1], y2, deg, b2.reshape(1, _F),
                    batch.reshape(1, _N).astype(jnp.int32),
                    Wil, bil.reshape(1, 64), Whl1, bhl1.reshape(1, 16),
                    Wol, bol.reshape(1, 1))
